# baseline (speedup 1.0000x reference)
import functools

import jax
import jax.numpy as jnp
from jax import lax
from jax.experimental import pallas as pl
from jax.experimental.pallas import tpu as pltpu
from jax.experimental.pallas import tpu_sc as plsc

T = 2048
D = 1024
F = 2048
E = 8
BLK = 256
NP = 4096 + E * BLK
NB = NP // BLK

_INTERPRET = False


def _router_body(x_ref, wg_ref, slot0_ref, slot1_ref, cw0_ref, cw1_ref, be_ref):
    x = x_ref[...]
    wg = wg_ref[...]
    dn = (((1,), (0,)), ((), ()))
    logits = lax.dot_general(x, wg, dn,
                             preferred_element_type=jnp.float32)
    mx = jnp.max(logits, axis=1, keepdims=True)
    unnorm = jnp.exp(logits - mx)
    probs = unnorm / jnp.sum(unnorm, axis=1, keepdims=True)
    eiota = lax.broadcasted_iota(jnp.int32, (T, E), 1)
    m1 = jnp.max(probs, axis=1, keepdims=True)
    idx1 = jnp.min(jnp.where(probs == m1, eiota, E), axis=1, keepdims=True)
    oh1 = eiota == idx1
    masked = jnp.where(oh1, -1.0, probs)
    m2 = jnp.max(masked, axis=1, keepdims=True)
    idx2 = jnp.min(jnp.where(masked == m2, eiota, E), axis=1, keepdims=True)
    oh2 = eiota == idx2
    s = m1 + m2
    cw0_ref[...] = jnp.broadcast_to(m1 / s, (T, 128))
    cw1_ref[...] = jnp.broadcast_to(m2 / s, (T, 128))
    oh1f = oh1.astype(jnp.float32)
    oh2f = oh2.astype(jnp.float32)
    tri = (lax.broadcasted_iota(jnp.int32, (T, T), 1)
           < lax.broadcasted_iota(jnp.int32, (T, T), 0)).astype(jnp.float32)
    p0 = lax.dot_general(tri, oh1f, dn,
                         preferred_element_type=jnp.float32)
    p1 = lax.dot_general(tri, oh2f, dn,
                         preferred_element_type=jnp.float32)
    cnt0 = jnp.sum(oh1f, axis=0, keepdims=True)
    cnt1 = jnp.sum(oh2f, axis=0, keepdims=True)
    counts = (cnt0 + cnt1).astype(jnp.int32)
    pci = (counts + (BLK - 1)) >> (BLK.bit_length() - 1)
    pcf = pci.astype(jnp.float32)
    upper = (lax.broadcasted_iota(jnp.int32, (E, E), 0)
             < lax.broadcasted_iota(jnp.int32, (E, E), 1)).astype(jnp.float32)
    base_blocks = lax.dot_general(pcf, upper, dn,
                                  preferred_element_type=jnp.float32)
    base = base_blocks * float(BLK)
    ub = base_blocks + pcf
    rank0 = jnp.sum(p0 * oh1f, axis=1, keepdims=True)
    rank1 = jnp.sum((p1 + cnt0) * oh2f, axis=1, keepdims=True)
    base0 = jnp.sum(base * oh1f, axis=1, keepdims=True)
    base1 = jnp.sum(base * oh2f, axis=1, keepdims=True)
    slot0_ref[...] = (base0 + rank0).astype(jnp.int32)
    slot1_ref[...] = (base1 + rank1).astype(jnp.int32)
    biota = lax.broadcasted_iota(jnp.int32, (NB, E), 0).astype(jnp.float32)
    own = jnp.sum((biota >= ub).astype(jnp.float32), axis=1, keepdims=True)
    eiota8 = lax.broadcasted_iota(jnp.int32, (1, E), 1).astype(jnp.float32)
    last_e = jnp.max(jnp.where(pcf > 0.0, eiota8, 0.0))
    own = jnp.minimum(own, last_e)
    used = jnp.sum(pcf)
    be_ref[...] = jnp.concatenate(
        [own, jnp.full((1, 1), used, jnp.float32)], axis=0).astype(jnp.int32)


def _router(x, wg):
    return pl.pallas_call(
        _router_body,
        out_shape=(
            jax.ShapeDtypeStruct((T, 1), jnp.int32),
            jax.ShapeDtypeStruct((T, 1), jnp.int32),
            jax.ShapeDtypeStruct((T, 128), jnp.float32),
            jax.ShapeDtypeStruct((T, 128), jnp.float32),
            jax.ShapeDtypeStruct((NB + 1, 1), jnp.int32),
        ),
        interpret=_INTERPRET,
    )(x, wg)


def _wcast_body(w1_ref, w2_ref, w1b_ref, w2b_ref):
    w1b_ref[...] = w1_ref[...].astype(jnp.bfloat16)
    w2b_ref[...] = w2_ref[...].astype(jnp.bfloat16)


def _wcast(w1, w2):
    return pl.pallas_call(
        _wcast_body,
        grid=(E,),
        in_specs=[
            pl.BlockSpec((1, D, F), lambda e: (e, 0, 0)),
            pl.BlockSpec((1, F, D), lambda e: (e, 0, 0)),
        ],
        out_specs=[
            pl.BlockSpec((1, D, F), lambda e: (e, 0, 0)),
            pl.BlockSpec((1, F, D), lambda e: (e, 0, 0)),
        ],
        out_shape=(
            jax.ShapeDtypeStruct((E, D, F), jnp.bfloat16),
            jax.ShapeDtypeStruct((E, F, D), jnp.bfloat16),
        ),
        interpret=_INTERPRET,
    )(w1, w2)


def _ffn_body(be_ref, xs_ref, w1_ref, w2_ref, ws_ref, ys_ref):
    @pl.when(pl.program_id(0) < be_ref[NB])
    def _():
        dn = (((1,), (0,)), ((), ()))
        xb = xs_ref[...].astype(jnp.bfloat16)
        h = lax.dot_general(xb, w1_ref[0], dn,
                            preferred_element_type=jnp.float32)
        h = jnp.maximum(h, 0.0).astype(jnp.bfloat16)
        y = lax.dot_general(h, w2_ref[0], dn,
                            preferred_element_type=jnp.float32)
        ys_ref[...] = y * ws_ref[:, 0:1]


def _ffn(be, xs, w1b, w2b, ws_col):
    grid_spec = pltpu.PrefetchScalarGridSpec(
        num_scalar_prefetch=1,
        grid=(NB,),
        in_specs=[
            pl.BlockSpec((BLK, D), lambda i, be_r: (i, 0)),
            pl.BlockSpec((1, D, F), lambda i, be_r: (be_r[i], 0, 0)),
            pl.BlockSpec((1, F, D), lambda i, be_r: (be_r[i], 0, 0)),
            pl.BlockSpec((BLK, 128), lambda i, be_r: (i, 0)),
        ],
        out_specs=pl.BlockSpec((BLK, D), lambda i, be_r: (i, 0)),
    )
    return pl.pallas_call(
        _ffn_body,
        grid_spec=grid_spec,
        out_shape=jax.ShapeDtypeStruct((NP, D), jnp.float32),
        compiler_params=pltpu.CompilerParams(
            dimension_semantics=("arbitrary",)),
        interpret=_INTERPRET,
    )(be, xs, w1b, w2b, ws_col)


_NW = 32
_TPW = T // _NW
_CCH = 32

_sc_mesh = plsc.VectorSubcoreMesh(core_axis_name="c", subcore_axis_name="s")


@functools.partial(
    pl.kernel,
    out_type=(
        jax.ShapeDtypeStruct((NP, D), jnp.float32),
        jax.ShapeDtypeStruct((NP, 128), jnp.float32),
    ),
    mesh=_sc_mesh,
    scratch_types=[
        pltpu.VMEM((_TPW,), jnp.int32),
        pltpu.VMEM((_TPW,), jnp.int32),
        pltpu.VMEM((_TPW, D), jnp.float32),
        pltpu.VMEM((_TPW, 128), jnp.float32),
        pltpu.VMEM((_TPW, 128), jnp.float32),
        pltpu.SemaphoreType.DMA,
        pltpu.SemaphoreType.DMA,
        pltpu.SemaphoreType.DMA,
        pltpu.SemaphoreType.DMA,
    ],
)
def _dispatch(pos0_hbm, pos1_hbm, cw0_hbm, cw1_hbm, x_hbm, xs_hbm, ws_hbm,
              p0_v, p1_v, xr_v, c0_v, c1_v, s1, s2, s3, s4):
    wid = lax.axis_index("s") * 2 + lax.axis_index("c")
    t0 = wid * _TPW
    pltpu.sync_copy(pos0_hbm.at[pl.ds(t0, _TPW)], p0_v)
    pltpu.sync_copy(pos1_hbm.at[pl.ds(t0, _TPW)], p1_v)
    pltpu.sync_copy(cw0_hbm.at[pl.ds(t0, _TPW)], c0_v)
    pltpu.sync_copy(cw1_hbm.at[pl.ds(t0, _TPW)], c1_v)
    pltpu.sync_copy(x_hbm.at[pl.ds(t0, _TPW)], xr_v)
    a = pltpu.async_copy(xr_v, xs_hbm.at[p0_v], s1)
    b = pltpu.async_copy(xr_v, xs_hbm.at[p1_v], s2)
    c = pltpu.async_copy(c0_v, ws_hbm.at[p0_v], s3)
    d = pltpu.async_copy(c1_v, ws_hbm.at[p1_v], s4)
    a.wait()
    b.wait()
    c.wait()
    d.wait()


@functools.partial(
    pl.kernel,
    out_type=jax.ShapeDtypeStruct((T, D), jnp.float32),
    mesh=_sc_mesh,
    scratch_types=[
        pltpu.VMEM((_TPW,), jnp.int32),
        pltpu.VMEM((_TPW,), jnp.int32),
        pltpu.VMEM((_CCH, D), jnp.float32),
        pltpu.VMEM((_CCH, D), jnp.float32),
        pltpu.SemaphoreType.DMA,
        pltpu.SemaphoreType.DMA,
    ],
)
def _combine(pos0_hbm, pos1_hbm, ys_hbm, out_hbm,
             p0_v, p1_v, ba_v, bb_v, sema, semb):
    wid = lax.axis_index("s") * 2 + lax.axis_index("c")
    t0 = wid * _TPW
    pltpu.sync_copy(pos0_hbm.at[pl.ds(t0, _TPW)], p0_v)
    pltpu.sync_copy(pos1_hbm.at[pl.ds(t0, _TPW)], p1_v)
    for cc in range(_TPW // _CCH):
        ca = pltpu.async_copy(ys_hbm.at[p0_v.at[pl.ds(cc * _CCH, _CCH)]],
                              ba_v, sema)
        cb = pltpu.async_copy(ys_hbm.at[p1_v.at[pl.ds(cc * _CCH, _CCH)]],
                              bb_v, semb)
        ca.wait()
        cb.wait()

        def add_row(r, carry):
            for j in range(D // 16):
                ba_v[r, pl.ds(j * 16, 16)] = (ba_v[r, pl.ds(j * 16, 16)]
                                              + bb_v[r, pl.ds(j * 16, 16)])
            return carry

        lax.fori_loop(0, _CCH, add_row, 0)
        pltpu.sync_copy(ba_v, out_hbm.at[pl.ds(t0 + cc * _CCH, _CCH)])


def kernel(x, Wg, w1, w2):
    slot0, slot1, cw0, cw1, be = _router(x, Wg)
    pos0 = slot0[:, 0]
    pos1 = slot1[:, 0]
    be_flat = be[:, 0]
    w1b, w2b = _wcast(w1, w2)
    xs, ws = _dispatch(pos0, pos1, cw0, cw1, x)
    ys = _ffn(be_flat, xs, w1b, w2b, ws)
    out = _combine(pos0, pos1, ys)
    return out


# speedup vs baseline: 1.2447x; 1.2447x over previous
import functools

import jax
import jax.numpy as jnp
from jax import lax
from jax.experimental import pallas as pl
from jax.experimental.pallas import tpu as pltpu
from jax.experimental.pallas import tpu_sc as plsc

T = 2048
D = 1024
F = 2048
E = 8
BLK = 256
NP = 4096 + E * BLK
NB = NP // BLK

_INTERPRET = False


def _router_body(x_ref, wg_ref, slot0_ref, slot1_ref, cw0_ref, cw1_ref, be_ref):
    x = x_ref[...]
    wg = wg_ref[...]
    dn = (((1,), (0,)), ((), ()))
    logits = lax.dot_general(x, wg, dn,
                             preferred_element_type=jnp.float32)
    mx = jnp.max(logits, axis=1, keepdims=True)
    unnorm = jnp.exp(logits - mx)
    probs = unnorm / jnp.sum(unnorm, axis=1, keepdims=True)
    eiota = lax.broadcasted_iota(jnp.int32, (T, E), 1)
    m1 = jnp.max(probs, axis=1, keepdims=True)
    idx1 = jnp.min(jnp.where(probs == m1, eiota, E), axis=1, keepdims=True)
    oh1 = eiota == idx1
    masked = jnp.where(oh1, -1.0, probs)
    m2 = jnp.max(masked, axis=1, keepdims=True)
    idx2 = jnp.min(jnp.where(masked == m2, eiota, E), axis=1, keepdims=True)
    oh2 = eiota == idx2
    s = m1 + m2
    cw0_ref[...] = jnp.broadcast_to(m1 / s, (T, 128))
    cw1_ref[...] = jnp.broadcast_to(m2 / s, (T, 128))
    oh1f = oh1.astype(jnp.float32)
    oh2f = oh2.astype(jnp.float32)
    tri = (lax.broadcasted_iota(jnp.int32, (T, T), 1)
           < lax.broadcasted_iota(jnp.int32, (T, T), 0)).astype(jnp.float32)
    p0 = lax.dot_general(tri, oh1f, dn,
                         preferred_element_type=jnp.float32)
    p1 = lax.dot_general(tri, oh2f, dn,
                         preferred_element_type=jnp.float32)
    cnt0 = jnp.sum(oh1f, axis=0, keepdims=True)
    cnt1 = jnp.sum(oh2f, axis=0, keepdims=True)
    counts = (cnt0 + cnt1).astype(jnp.int32)
    pci = (counts + (BLK - 1)) >> (BLK.bit_length() - 1)
    pcf = pci.astype(jnp.float32)
    upper = (lax.broadcasted_iota(jnp.int32, (E, E), 0)
             < lax.broadcasted_iota(jnp.int32, (E, E), 1)).astype(jnp.float32)
    base_blocks = lax.dot_general(pcf, upper, dn,
                                  preferred_element_type=jnp.float32)
    base = base_blocks * float(BLK)
    ub = base_blocks + pcf
    rank0 = jnp.sum(p0 * oh1f, axis=1, keepdims=True)
    rank1 = jnp.sum((p1 + cnt0) * oh2f, axis=1, keepdims=True)
    base0 = jnp.sum(base * oh1f, axis=1, keepdims=True)
    base1 = jnp.sum(base * oh2f, axis=1, keepdims=True)
    slot0_ref[...] = (base0 + rank0).astype(jnp.int32)
    slot1_ref[...] = (base1 + rank1).astype(jnp.int32)
    biota = lax.broadcasted_iota(jnp.int32, (NB, E), 0).astype(jnp.float32)
    own = jnp.sum((biota >= ub).astype(jnp.float32), axis=1, keepdims=True)
    eiota8 = lax.broadcasted_iota(jnp.int32, (1, E), 1).astype(jnp.float32)
    last_e = jnp.max(jnp.where(pcf > 0.0, eiota8, 0.0))
    own = jnp.minimum(own, last_e)
    used = jnp.sum(pcf)
    be_ref[...] = jnp.concatenate(
        [own, jnp.full((1, 1), used, jnp.float32)], axis=0).astype(jnp.int32)


def _router(x, wg):
    return pl.pallas_call(
        _router_body,
        out_shape=(
            jax.ShapeDtypeStruct((T, 1), jnp.int32),
            jax.ShapeDtypeStruct((T, 1), jnp.int32),
            jax.ShapeDtypeStruct((T, 128), jnp.float32),
            jax.ShapeDtypeStruct((T, 128), jnp.float32),
            jax.ShapeDtypeStruct((NB + 1, 1), jnp.int32),
        ),
        interpret=_INTERPRET,
    )(x, wg)


def _ffn_body(be_ref, xs_ref, w1_ref, w2_ref, ws_ref, ys_ref):
    @pl.when(pl.program_id(0) < be_ref[NB])
    def _():
        dn = (((1,), (0,)), ((), ()))
        h = lax.dot_general(xs_ref[...], w1_ref[0], dn,
                            preferred_element_type=jnp.float32)
        h = jnp.maximum(h, 0.0)
        y = lax.dot_general(h, w2_ref[0], dn,
                            preferred_element_type=jnp.float32)
        ys_ref[...] = y * ws_ref[:, 0:1]


def _ffn(be, xs, w1b, w2b, ws_col):
    grid_spec = pltpu.PrefetchScalarGridSpec(
        num_scalar_prefetch=1,
        grid=(NB,),
        in_specs=[
            pl.BlockSpec((BLK, D), lambda i, be_r: (i, 0)),
            pl.BlockSpec((1, D, F), lambda i, be_r: (be_r[i], 0, 0)),
            pl.BlockSpec((1, F, D), lambda i, be_r: (be_r[i], 0, 0)),
            pl.BlockSpec((BLK, 128), lambda i, be_r: (i, 0)),
        ],
        out_specs=pl.BlockSpec((BLK, D), lambda i, be_r: (i, 0)),
    )
    return pl.pallas_call(
        _ffn_body,
        grid_spec=grid_spec,
        out_shape=jax.ShapeDtypeStruct((NP, D), jnp.float32),
        compiler_params=pltpu.CompilerParams(
            dimension_semantics=("arbitrary",)),
        interpret=_INTERPRET,
    )(be, xs, w1b, w2b, ws_col)


_NW = 32
_TPW = T // _NW
_CCH = 32

_sc_mesh = plsc.VectorSubcoreMesh(core_axis_name="c", subcore_axis_name="s")


@functools.partial(
    pl.kernel,
    out_type=(
        jax.ShapeDtypeStruct((NP, D), jnp.float32),
        jax.ShapeDtypeStruct((NP, 128), jnp.float32),
    ),
    mesh=_sc_mesh,
    scratch_types=[
        pltpu.VMEM((_TPW,), jnp.int32),
        pltpu.VMEM((_TPW,), jnp.int32),
        pltpu.VMEM((_TPW, D), jnp.float32),
        pltpu.VMEM((_TPW, 128), jnp.float32),
        pltpu.VMEM((_TPW, 128), jnp.float32),
        pltpu.SemaphoreType.DMA,
        pltpu.SemaphoreType.DMA,
        pltpu.SemaphoreType.DMA,
        pltpu.SemaphoreType.DMA,
    ],
)
def _dispatch(pos0_hbm, pos1_hbm, cw0_hbm, cw1_hbm, x_hbm, xs_hbm, ws_hbm,
              p0_v, p1_v, xr_v, c0_v, c1_v, s1, s2, s3, s4):
    wid = lax.axis_index("s") * 2 + lax.axis_index("c")
    t0 = wid * _TPW
    pltpu.sync_copy(pos0_hbm.at[pl.ds(t0, _TPW)], p0_v)
    pltpu.sync_copy(pos1_hbm.at[pl.ds(t0, _TPW)], p1_v)
    pltpu.sync_copy(cw0_hbm.at[pl.ds(t0, _TPW)], c0_v)
    pltpu.sync_copy(cw1_hbm.at[pl.ds(t0, _TPW)], c1_v)
    pltpu.sync_copy(x_hbm.at[pl.ds(t0, _TPW)], xr_v)
    a = pltpu.async_copy(xr_v, xs_hbm.at[p0_v], s1)
    b = pltpu.async_copy(xr_v, xs_hbm.at[p1_v], s2)
    c = pltpu.async_copy(c0_v, ws_hbm.at[p0_v], s3)
    d = pltpu.async_copy(c1_v, ws_hbm.at[p1_v], s4)
    a.wait()
    b.wait()
    c.wait()
    d.wait()


@functools.partial(
    pl.kernel,
    out_type=jax.ShapeDtypeStruct((T, D), jnp.float32),
    mesh=_sc_mesh,
    scratch_types=[
        pltpu.VMEM((_TPW,), jnp.int32),
        pltpu.VMEM((_TPW,), jnp.int32),
        pltpu.VMEM((_CCH, D), jnp.float32),
        pltpu.VMEM((_CCH, D), jnp.float32),
        pltpu.SemaphoreType.DMA,
        pltpu.SemaphoreType.DMA,
    ],
)
def _combine(pos0_hbm, pos1_hbm, ys_hbm, out_hbm,
             p0_v, p1_v, ba_v, bb_v, sema, semb):
    wid = lax.axis_index("s") * 2 + lax.axis_index("c")
    t0 = wid * _TPW
    pltpu.sync_copy(pos0_hbm.at[pl.ds(t0, _TPW)], p0_v)
    pltpu.sync_copy(pos1_hbm.at[pl.ds(t0, _TPW)], p1_v)
    for cc in range(_TPW // _CCH):
        ca = pltpu.async_copy(ys_hbm.at[p0_v.at[pl.ds(cc * _CCH, _CCH)]],
                              ba_v, sema)
        cb = pltpu.async_copy(ys_hbm.at[p1_v.at[pl.ds(cc * _CCH, _CCH)]],
                              bb_v, semb)
        ca.wait()
        cb.wait()

        def add_row(r, carry):
            for j in range(D // 16):
                ba_v[r, pl.ds(j * 16, 16)] = (ba_v[r, pl.ds(j * 16, 16)]
                                              + bb_v[r, pl.ds(j * 16, 16)])
            return carry

        lax.fori_loop(0, _CCH, add_row, 0)
        pltpu.sync_copy(ba_v, out_hbm.at[pl.ds(t0 + cc * _CCH, _CCH)])


def kernel(x, Wg, w1, w2):
    slot0, slot1, cw0, cw1, be = _router(x, Wg)
    pos0 = slot0[:, 0]
    pos1 = slot1[:, 0]
    be_flat = be[:, 0]
    xs, ws = _dispatch(pos0, pos1, cw0, cw1, x)
    ys = _ffn(be_flat, xs, w1, w2, ws)
    out = _combine(pos0, pos1, ys)
    return out


# speedup vs baseline: 1.2588x; 1.0113x over previous
import functools

import jax
import jax.numpy as jnp
from jax import lax
from jax.experimental import pallas as pl
from jax.experimental.pallas import tpu as pltpu
from jax.experimental.pallas import tpu_sc as plsc

T = 2048
D = 1024
F = 2048
E = 8
BLK = 256
NP = 4096 + E * BLK
NB = NP // BLK

_INTERPRET = False


def _router_body(x_ref, wg_ref, slot0_ref, slot1_ref, cw0_ref, cw1_ref, be_ref):
    x = x_ref[...]
    wg = wg_ref[...]
    dn = (((1,), (0,)), ((), ()))
    logits = lax.dot_general(x, wg, dn,
                             preferred_element_type=jnp.float32)
    mx = jnp.max(logits, axis=1, keepdims=True)
    unnorm = jnp.exp(logits - mx)
    probs = unnorm / jnp.sum(unnorm, axis=1, keepdims=True)
    eiota = lax.broadcasted_iota(jnp.int32, (T, E), 1)
    m1 = jnp.max(probs, axis=1, keepdims=True)
    idx1 = jnp.min(jnp.where(probs == m1, eiota, E), axis=1, keepdims=True)
    oh1 = eiota == idx1
    masked = jnp.where(oh1, -1.0, probs)
    m2 = jnp.max(masked, axis=1, keepdims=True)
    idx2 = jnp.min(jnp.where(masked == m2, eiota, E), axis=1, keepdims=True)
    oh2 = eiota == idx2
    s = m1 + m2
    cw0_ref[...] = jnp.broadcast_to(m1 / s, (T, 128))
    cw1_ref[...] = jnp.broadcast_to(m2 / s, (T, 128))
    oh1f = oh1.astype(jnp.float32)
    oh2f = oh2.astype(jnp.float32)
    tri = (lax.broadcasted_iota(jnp.int32, (T, T), 1)
           < lax.broadcasted_iota(jnp.int32, (T, T), 0)).astype(jnp.float32)
    p0 = lax.dot_general(tri, oh1f, dn,
                         preferred_element_type=jnp.float32)
    p1 = lax.dot_general(tri, oh2f, dn,
                         preferred_element_type=jnp.float32)
    cnt0 = jnp.sum(oh1f, axis=0, keepdims=True)
    cnt1 = jnp.sum(oh2f, axis=0, keepdims=True)
    counts = (cnt0 + cnt1).astype(jnp.int32)
    pci = (counts + (BLK - 1)) >> (BLK.bit_length() - 1)
    pcf = pci.astype(jnp.float32)
    upper = (lax.broadcasted_iota(jnp.int32, (E, E), 0)
             < lax.broadcasted_iota(jnp.int32, (E, E), 1)).astype(jnp.float32)
    base_blocks = lax.dot_general(pcf, upper, dn,
                                  preferred_element_type=jnp.float32)
    base = base_blocks * float(BLK)
    ub = base_blocks + pcf
    rank0 = jnp.sum(p0 * oh1f, axis=1, keepdims=True)
    rank1 = jnp.sum((p1 + cnt0) * oh2f, axis=1, keepdims=True)
    base0 = jnp.sum(base * oh1f, axis=1, keepdims=True)
    base1 = jnp.sum(base * oh2f, axis=1, keepdims=True)
    slot0_ref[...] = (base0 + rank0).astype(jnp.int32)
    slot1_ref[...] = (base1 + rank1).astype(jnp.int32)
    biota = lax.broadcasted_iota(jnp.int32, (NB, E), 0).astype(jnp.float32)
    own = jnp.sum((biota >= ub).astype(jnp.float32), axis=1, keepdims=True)
    eiota8 = lax.broadcasted_iota(jnp.int32, (1, E), 1).astype(jnp.float32)
    last_e = jnp.max(jnp.where(pcf > 0.0, eiota8, 0.0))
    own = jnp.minimum(own, last_e)
    used = jnp.sum(pcf)
    be_ref[...] = jnp.concatenate(
        [own, jnp.full((1, 1), used, jnp.float32)], axis=0).astype(jnp.int32)


def _router(x, wg):
    return pl.pallas_call(
        _router_body,
        out_shape=(
            jax.ShapeDtypeStruct((T, 1), jnp.int32),
            jax.ShapeDtypeStruct((T, 1), jnp.int32),
            jax.ShapeDtypeStruct((T, 128), jnp.float32),
            jax.ShapeDtypeStruct((T, 128), jnp.float32),
            jax.ShapeDtypeStruct((NB + 1, 1), jnp.int32),
        ),
        interpret=_INTERPRET,
    )(x, wg)


def _ffn_body(be_ref, xs_ref, w1_ref, w2_ref, ws_ref, ys_ref):
    @pl.when(pl.program_id(0) < be_ref[NB])
    def _():
        dn = (((1,), (0,)), ((), ()))
        h = lax.dot_general(xs_ref[...], w1_ref[0], dn,
                            preferred_element_type=jnp.float32)
        h = jnp.maximum(h, 0.0)
        y = lax.dot_general(h, w2_ref[0], dn,
                            preferred_element_type=jnp.float32)
        ys_ref[...] = y * ws_ref[:, 0:1]


def _ffn(be, xs, w1b, w2b, ws_col):
    grid_spec = pltpu.PrefetchScalarGridSpec(
        num_scalar_prefetch=1,
        grid=(NB,),
        in_specs=[
            pl.BlockSpec((BLK, D), lambda i, be_r: (i, 0)),
            pl.BlockSpec((1, D, F), lambda i, be_r: (be_r[i], 0, 0)),
            pl.BlockSpec((1, F, D), lambda i, be_r: (be_r[i], 0, 0)),
            pl.BlockSpec((BLK, 128), lambda i, be_r: (i, 0)),
        ],
        out_specs=pl.BlockSpec((BLK, D), lambda i, be_r: (i, 0)),
    )
    return pl.pallas_call(
        _ffn_body,
        grid_spec=grid_spec,
        out_shape=jax.ShapeDtypeStruct((NP, D), jnp.float32),
        compiler_params=pltpu.CompilerParams(
            dimension_semantics=("arbitrary",)),
        interpret=_INTERPRET,
    )(be, xs, w1b, w2b, ws_col)


_NW = 32
_TPW = T // _NW
_CCH = 16

_sc_mesh = plsc.VectorSubcoreMesh(core_axis_name="c", subcore_axis_name="s")


@functools.partial(
    pl.kernel,
    out_type=(
        jax.ShapeDtypeStruct((NP, D), jnp.float32),
        jax.ShapeDtypeStruct((NP, 128), jnp.float32),
    ),
    mesh=_sc_mesh,
    scratch_types=[
        pltpu.VMEM((_TPW,), jnp.int32),
        pltpu.VMEM((_TPW,), jnp.int32),
        pltpu.VMEM((_TPW, D), jnp.float32),
        pltpu.VMEM((_TPW, 128), jnp.float32),
        pltpu.VMEM((_TPW, 128), jnp.float32),
        pltpu.SemaphoreType.DMA,
        pltpu.SemaphoreType.DMA,
        pltpu.SemaphoreType.DMA,
        pltpu.SemaphoreType.DMA,
        pltpu.SemaphoreType.DMA,
    ],
)
def _dispatch(pos0_hbm, pos1_hbm, cw0_hbm, cw1_hbm, x_hbm, xs_hbm, ws_hbm,
              p0_v, p1_v, xr_v, c0_v, c1_v, s1, s2, s3, s4, s5):
    wid = lax.axis_index("s") * 2 + lax.axis_index("c")
    t0 = wid * _TPW
    l1 = pltpu.async_copy(pos0_hbm.at[pl.ds(t0, _TPW)], p0_v, s1)
    l2 = pltpu.async_copy(pos1_hbm.at[pl.ds(t0, _TPW)], p1_v, s2)
    l3 = pltpu.async_copy(cw0_hbm.at[pl.ds(t0, _TPW)], c0_v, s3)
    l4 = pltpu.async_copy(cw1_hbm.at[pl.ds(t0, _TPW)], c1_v, s4)
    l5 = pltpu.async_copy(x_hbm.at[pl.ds(t0, _TPW)], xr_v, s5)
    l1.wait()
    l2.wait()
    l3.wait()
    l4.wait()
    l5.wait()
    a = pltpu.async_copy(xr_v, xs_hbm.at[p0_v], s1)
    b = pltpu.async_copy(xr_v, xs_hbm.at[p1_v], s2)
    c = pltpu.async_copy(c0_v, ws_hbm.at[p0_v], s3)
    d = pltpu.async_copy(c1_v, ws_hbm.at[p1_v], s4)
    a.wait()
    b.wait()
    c.wait()
    d.wait()


@functools.partial(
    pl.kernel,
    out_type=jax.ShapeDtypeStruct((T, D), jnp.float32),
    mesh=_sc_mesh,
    scratch_types=[
        pltpu.VMEM((_TPW,), jnp.int32),
        pltpu.VMEM((_TPW,), jnp.int32),
        pltpu.VMEM((2, _CCH, D), jnp.float32),
        pltpu.VMEM((2, _CCH, D), jnp.float32),
        pltpu.SemaphoreType.DMA,
        pltpu.SemaphoreType.DMA,
        pltpu.SemaphoreType.DMA,
        pltpu.SemaphoreType.DMA,
    ],
)
def _combine(pos0_hbm, pos1_hbm, ys_hbm, out_hbm,
             p0_v, p1_v, ba_v, bb_v, sa0, sa1, sb0, sb1):
    wid = lax.axis_index("s") * 2 + lax.axis_index("c")
    t0 = wid * _TPW
    pltpu.sync_copy(pos0_hbm.at[pl.ds(t0, _TPW)], p0_v)
    pltpu.sync_copy(pos1_hbm.at[pl.ds(t0, _TPW)], p1_v)
    ncc = _TPW // _CCH
    sas = (sa0, sa1)
    sbs = (sb0, sb1)

    def fire(cc):
        pb = cc % 2
        ca = pltpu.async_copy(ys_hbm.at[p0_v.at[pl.ds(cc * _CCH, _CCH)]],
                              ba_v.at[pb], sas[pb])
        cb = pltpu.async_copy(ys_hbm.at[p1_v.at[pl.ds(cc * _CCH, _CCH)]],
                              bb_v.at[pb], sbs[pb])
        return ca, cb

    pend = fire(0)
    for cc in range(ncc):
        pb = cc % 2
        nxt = fire(cc + 1) if cc + 1 < ncc else None
        pend[0].wait()
        pend[1].wait()

        def add_row(r, carry):
            for j in range(D // 16):
                ba_v[pb, r, pl.ds(j * 16, 16)] = (
                    ba_v[pb, r, pl.ds(j * 16, 16)]
                    + bb_v[pb, r, pl.ds(j * 16, 16)])
            return carry

        lax.fori_loop(0, _CCH, add_row, 0)
        pltpu.sync_copy(ba_v.at[pb], out_hbm.at[pl.ds(t0 + cc * _CCH, _CCH)])
        pend = nxt


def kernel(x, Wg, w1, w2):
    slot0, slot1, cw0, cw1, be = _router(x, Wg)
    pos0 = slot0[:, 0]
    pos1 = slot1[:, 0]
    be_flat = be[:, 0]
    xs, ws = _dispatch(pos0, pos1, cw0, cw1, x)
    ys = _ffn(be_flat, xs, w1, w2, ws)
    out = _combine(pos0, pos1, ys)
    return out


# speedup vs baseline: 1.3030x; 1.0351x over previous
import functools

import jax
import jax.numpy as jnp
from jax import lax
from jax.experimental import pallas as pl
from jax.experimental.pallas import tpu as pltpu
from jax.experimental.pallas import tpu_sc as plsc

T = 2048
D = 1024
F = 2048
E = 8
BLK = 256
NP = 4096 + E * BLK
NB = NP // BLK

_INTERPRET = False


def _router_body(x_ref, wg_ref, slot0_ref, slot1_ref, cw0_ref, cw1_ref, be_ref):
    x = x_ref[...]
    wg = wg_ref[...]
    dn = (((1,), (0,)), ((), ()))
    logits = lax.dot_general(x, wg, dn,
                             preferred_element_type=jnp.float32)
    mx = jnp.max(logits, axis=1, keepdims=True)
    unnorm = jnp.exp(logits - mx)
    probs = unnorm / jnp.sum(unnorm, axis=1, keepdims=True)
    eiota = lax.broadcasted_iota(jnp.int32, (T, E), 1)
    m1 = jnp.max(probs, axis=1, keepdims=True)
    idx1 = jnp.min(jnp.where(probs == m1, eiota, E), axis=1, keepdims=True)
    oh1 = eiota == idx1
    masked = jnp.where(oh1, -1.0, probs)
    m2 = jnp.max(masked, axis=1, keepdims=True)
    idx2 = jnp.min(jnp.where(masked == m2, eiota, E), axis=1, keepdims=True)
    oh2 = eiota == idx2
    s = m1 + m2
    cw0_ref[...] = jnp.broadcast_to(m1 / s, (T, 128))
    cw1_ref[...] = jnp.broadcast_to(m2 / s, (T, 128))
    oh1f = oh1.astype(jnp.float32)
    oh2f = oh2.astype(jnp.float32)
    tri = (lax.broadcasted_iota(jnp.int32, (T, T), 1)
           < lax.broadcasted_iota(jnp.int32, (T, T), 0)).astype(jnp.float32)
    p0 = lax.dot_general(tri, oh1f, dn,
                         preferred_element_type=jnp.float32)
    p1 = lax.dot_general(tri, oh2f, dn,
                         preferred_element_type=jnp.float32)
    cnt0 = jnp.sum(oh1f, axis=0, keepdims=True)
    cnt1 = jnp.sum(oh2f, axis=0, keepdims=True)
    counts = (cnt0 + cnt1).astype(jnp.int32)
    pci = (counts + (BLK - 1)) >> (BLK.bit_length() - 1)
    pcf = pci.astype(jnp.float32)
    upper = (lax.broadcasted_iota(jnp.int32, (E, E), 0)
             < lax.broadcasted_iota(jnp.int32, (E, E), 1)).astype(jnp.float32)
    base_blocks = lax.dot_general(pcf, upper, dn,
                                  preferred_element_type=jnp.float32)
    base = base_blocks * float(BLK)
    ub = base_blocks + pcf
    rank0 = jnp.sum(p0 * oh1f, axis=1, keepdims=True)
    rank1 = jnp.sum((p1 + cnt0) * oh2f, axis=1, keepdims=True)
    base0 = jnp.sum(base * oh1f, axis=1, keepdims=True)
    base1 = jnp.sum(base * oh2f, axis=1, keepdims=True)
    slot0_ref[...] = (base0 + rank0).astype(jnp.int32)
    slot1_ref[...] = (base1 + rank1).astype(jnp.int32)
    biota = lax.broadcasted_iota(jnp.int32, (NB, E), 0).astype(jnp.float32)
    own = jnp.sum((biota >= ub).astype(jnp.float32), axis=1, keepdims=True)
    eiota8 = lax.broadcasted_iota(jnp.int32, (1, E), 1).astype(jnp.float32)
    last_e = jnp.max(jnp.where(pcf > 0.0, eiota8, 0.0))
    own = jnp.minimum(own, last_e)
    used = jnp.sum(pcf)
    be_ref[...] = jnp.concatenate(
        [own, jnp.full((1, 1), used, jnp.float32)], axis=0).astype(jnp.int32)


def _router(x, wg):
    return pl.pallas_call(
        _router_body,
        out_shape=(
            jax.ShapeDtypeStruct((T, 1), jnp.int32),
            jax.ShapeDtypeStruct((T, 1), jnp.int32),
            jax.ShapeDtypeStruct((T, 128), jnp.float32),
            jax.ShapeDtypeStruct((T, 128), jnp.float32),
            jax.ShapeDtypeStruct((NB + 1, 1), jnp.int32),
        ),
        interpret=_INTERPRET,
    )(x, wg)


def _ffn_body(be_ref, xs_ref, w1_ref, w2_ref, ws_ref, ys_ref):
    @pl.when(pl.program_id(0) < be_ref[NB])
    def _():
        dn = (((1,), (0,)), ((), ()))
        h = lax.dot_general(xs_ref[...], w1_ref[0], dn,
                            preferred_element_type=jnp.float32)
        h = jnp.maximum(h, 0.0)
        y = lax.dot_general(h, w2_ref[0], dn,
                            preferred_element_type=jnp.float32)
        ys_ref[...] = y * ws_ref[:, 0:1]


def _ffn(be, xs, w1b, w2b, ws_col):
    grid_spec = pltpu.PrefetchScalarGridSpec(
        num_scalar_prefetch=1,
        grid=(NB,),
        in_specs=[
            pl.BlockSpec((BLK, D), lambda i, be_r: (i, 0)),
            pl.BlockSpec((1, D, F), lambda i, be_r: (be_r[i], 0, 0)),
            pl.BlockSpec((1, F, D), lambda i, be_r: (be_r[i], 0, 0)),
            pl.BlockSpec((BLK, 128), lambda i, be_r: (i, 0)),
        ],
        out_specs=pl.BlockSpec((BLK, D), lambda i, be_r: (i, 0)),
    )
    return pl.pallas_call(
        _ffn_body,
        grid_spec=grid_spec,
        out_shape=jax.ShapeDtypeStruct((NP, D), jnp.float32),
        compiler_params=pltpu.CompilerParams(
            dimension_semantics=("arbitrary",)),
        interpret=_INTERPRET,
    )(be, xs, w1b, w2b, ws_col)


_NW = 32
_TPW = T // _NW
_CCH = 16

_sc_mesh = plsc.VectorSubcoreMesh(core_axis_name="c", subcore_axis_name="s")


@functools.partial(
    pl.kernel,
    out_type=(
        jax.ShapeDtypeStruct((NP, D), jnp.float32),
        jax.ShapeDtypeStruct((NP, 128), jnp.float32),
    ),
    mesh=_sc_mesh,
    scratch_types=[
        pltpu.VMEM((_TPW,), jnp.int32),
        pltpu.VMEM((_TPW,), jnp.int32),
        pltpu.VMEM((_TPW, D), jnp.float32),
        pltpu.VMEM((_TPW, 128), jnp.float32),
        pltpu.VMEM((_TPW, 128), jnp.float32),
        pltpu.SemaphoreType.DMA,
        pltpu.SemaphoreType.DMA,
        pltpu.SemaphoreType.DMA,
        pltpu.SemaphoreType.DMA,
        pltpu.SemaphoreType.DMA,
    ],
)
def _dispatch(pos0_hbm, pos1_hbm, cw0_hbm, cw1_hbm, x_hbm, xs_hbm, ws_hbm,
              p0_v, p1_v, xr_v, c0_v, c1_v, s1, s2, s3, s4, s5):
    wid = lax.axis_index("s") * 2 + lax.axis_index("c")
    t0 = wid * _TPW
    l1 = pltpu.async_copy(pos0_hbm.at[pl.ds(t0, _TPW)], p0_v, s1)
    l2 = pltpu.async_copy(pos1_hbm.at[pl.ds(t0, _TPW)], p1_v, s2)
    l3 = pltpu.async_copy(cw0_hbm.at[pl.ds(t0, _TPW)], c0_v, s3)
    l4 = pltpu.async_copy(cw1_hbm.at[pl.ds(t0, _TPW)], c1_v, s4)
    l5 = pltpu.async_copy(x_hbm.at[pl.ds(t0, _TPW)], xr_v, s5)
    l1.wait()
    l2.wait()
    l3.wait()
    l4.wait()
    l5.wait()
    a = pltpu.async_copy(xr_v, xs_hbm.at[p0_v], s1)
    b = pltpu.async_copy(xr_v, xs_hbm.at[p1_v], s2)
    c = pltpu.async_copy(c0_v, ws_hbm.at[p0_v], s3)
    d = pltpu.async_copy(c1_v, ws_hbm.at[p1_v], s4)
    a.wait()
    b.wait()
    c.wait()
    d.wait()


@functools.partial(
    pl.kernel,
    out_type=jax.ShapeDtypeStruct((T, D), jnp.float32),
    mesh=_sc_mesh,
    scratch_types=[
        pltpu.VMEM((_TPW,), jnp.int32),
        pltpu.VMEM((_TPW,), jnp.int32),
        pltpu.VMEM((2, _CCH, D), jnp.float32),
        pltpu.VMEM((2, _CCH, D), jnp.float32),
        pltpu.SemaphoreType.DMA,
        pltpu.SemaphoreType.DMA,
        pltpu.SemaphoreType.DMA,
        pltpu.SemaphoreType.DMA,
    ],
)
def _combine(pos0_hbm, pos1_hbm, ys_hbm, out_hbm,
             p0_v, p1_v, ba_v, bb_v, sa0, sa1, sb0, sb1):
    wid = lax.axis_index("s") * 2 + lax.axis_index("c")
    t0 = wid * _TPW
    pltpu.sync_copy(pos0_hbm.at[pl.ds(t0, _TPW)], p0_v)
    pltpu.sync_copy(pos1_hbm.at[pl.ds(t0, _TPW)], p1_v)
    ncc = _TPW // _CCH
    sas = (sa0, sa1)
    sbs = (sb0, sb1)

    def fire(cc):
        pb = cc % 2
        ca = pltpu.async_copy(ys_hbm.at[p0_v.at[pl.ds(cc * _CCH, _CCH)]],
                              ba_v.at[pb], sas[pb])
        cb = pltpu.async_copy(ys_hbm.at[p1_v.at[pl.ds(cc * _CCH, _CCH)]],
                              bb_v.at[pb], sbs[pb])
        return ca, cb

    pend = fire(0)
    for cc in range(ncc):
        pb = cc % 2
        nxt = fire(cc + 1) if cc + 1 < ncc else None
        pend[0].wait()
        pend[1].wait()

        @plsc.parallel_loop(0, _CCH * (D // 16), unroll=4)
        def _(k):
            r = k // (D // 16)
            j = k % (D // 16)
            ba_v[pb, r, pl.ds(j * 16, 16)] = (
                ba_v[pb, r, pl.ds(j * 16, 16)]
                + bb_v[pb, r, pl.ds(j * 16, 16)])
        pltpu.sync_copy(ba_v.at[pb], out_hbm.at[pl.ds(t0 + cc * _CCH, _CCH)])
        pend = nxt


def kernel(x, Wg, w1, w2):
    slot0, slot1, cw0, cw1, be = _router(x, Wg)
    pos0 = slot0[:, 0]
    pos1 = slot1[:, 0]
    be_flat = be[:, 0]
    xs, ws = _dispatch(pos0, pos1, cw0, cw1, x)
    ys = _ffn(be_flat, xs, w1, w2, ws)
    out = _combine(pos0, pos1, ys)
    return out
